# baseline (speedup 1.0000x reference)
import functools

import jax
import jax.numpy as jnp
from jax.experimental import pallas as pl


def _lrelu(v):
    return jnp.where(v > 0, v, 0.1 * v)


def _dot(a, b):
    return jax.lax.dot_general(
        a, b, (((1,), (0,)), ((), ())), preferred_element_type=jnp.float32
    )


def _t_kernel(x_ref, o_ref):
    o_ref[0] = jnp.transpose(x_ref[0], (1, 2, 0))


def _to_nhwc(x):
    B, C, H, W = x.shape
    R = 16
    return pl.pallas_call(
        _t_kernel,
        grid=(B, H // R),
        in_specs=[pl.BlockSpec((1, C, R, W), lambda b, i: (b, 0, i, 0))],
        out_specs=pl.BlockSpec((1, R, W, C), lambda b, i: (b, i, 0, 0)),
        out_shape=jax.ShapeDtypeStruct((B, H, W, C), jnp.float32),
    )(x)


def _s2_kernel(xe_ref, xeh1_ref, xeh2_ref, xo_ref, xoh1_ref, w_ref, o_ref,
               *, T, Wq, R, W2):
    xe = jnp.concatenate([xe_ref[0], xeh1_ref[0], xeh2_ref[0]], axis=0)
    xo = jnp.concatenate([xo_ref[0], xoh1_ref[0]], axis=0)
    acc = _dot(xe[0:T], w_ref[0])
    acc = acc + _dot(xe[1 : T + 1], w_ref[1])
    acc = acc + _dot(xo[0:T], w_ref[2])
    acc = acc + _dot(xo[1 : T + 1], w_ref[3])
    acc = acc + _dot(xe[Wq : Wq + T], w_ref[4])
    acc = acc + _dot(xe[Wq + 1 : Wq + T + 1], w_ref[5])
    acc = _lrelu(acc).reshape(R, Wq, -1)
    o_ref[0] = acc[:, :W2, :]


def _w_s2(w):
    O, C, _, _ = w.shape
    t = jnp.pad(w, ((0, 0), (0, 0), (0, 0), (0, 1)))
    t = t.reshape(O, C, 3, 2, 2)
    return t.transpose(2, 3, 4, 1, 0).reshape(6, 2 * C, O)


def _conv3x3_s2(x, w, R):
    B, H, W, C = x.shape
    O = w.shape[0]
    H2, W2 = H // 2, W // 2
    Wq = -(-(W2 + 1) // 8) * 8
    C2 = 2 * C
    f = x.reshape(B, H, W2, C2)
    xe = jnp.pad(f[:, 0::2], ((0, 0), (0, 2), (0, Wq - W2), (0, 0)))
    xo = jnp.pad(f[:, 1::2], ((0, 0), (0, 2), (0, Wq - W2), (0, 0)))
    xe = xe.reshape(B, (H2 + 2) * Wq, C2)
    xo = xo.reshape(B, (H2 + 2) * Wq, C2)
    w6 = _w_s2(w)
    T = R * Wq

    def _halo(j):
        return pl.BlockSpec(
            (1, Wq, C2),
            functools.partial(lambda b, i, jj: (b, (i + 1) * R + jj, 0), jj=j),
        )

    main = pl.BlockSpec((1, T, C2), lambda b, i: (b, i, 0))
    kfn = functools.partial(_s2_kernel, T=T, Wq=Wq, R=R, W2=W2)
    return pl.pallas_call(
        kfn,
        grid=(B, H2 // R),
        in_specs=[main, _halo(0), _halo(1), main, _halo(0),
                  pl.BlockSpec((6, C2, O), lambda b, i: (0, 0, 0))],
        out_specs=pl.BlockSpec((1, R, W2, O), lambda b, i: (b, i, 0, 0)),
        out_shape=jax.ShapeDtypeStruct((B, H2, W2, O), jnp.float32),
    )(xe, xe, xe, xo, xo, w6)


def _s1_kernel(x_ref, wf_ref, wh_ref, o_ref, *, T, offs):
    xs = x_ref[0]
    acc = None
    for k, off in enumerate(offs):
        t = _dot(xs[off : off + T, :], wf_ref[k])
        acc = t if acc is None else acc + t
    acc = _lrelu(acc)
    o_ref[0] = _dot(acc, wh_ref[...])


def _conv3x3_s1_head(x, wf, wh):
    B, H, W, C = x.shape
    Wp = -(-(W + 2) // 8) * 8
    xp = jnp.pad(x, ((0, 0), (1, 2), (1, Wp - W - 1), (0, 0)))
    xf = xp.reshape(B, (H + 3) * Wp, C)
    Npad = xf.shape[1]
    wft = jnp.transpose(wf, (2, 3, 1, 0)).reshape(9, C, wf.shape[0])
    wht = jnp.transpose(wh[:, :, 0, 0], (1, 0))
    Oh = wh.shape[0]
    offs = [dy * Wp + dx for dy in range(3) for dx in range(3)]
    T = H * Wp
    kfn = functools.partial(_s1_kernel, T=T, offs=offs)
    out = pl.pallas_call(
        kfn,
        grid=(B,),
        in_specs=[
            pl.BlockSpec((1, Npad, C), lambda b: (b, 0, 0)),
            pl.BlockSpec((9, C, wf.shape[0]), lambda b: (0, 0, 0)),
            pl.BlockSpec((C, Oh), lambda b: (0, 0)),
        ],
        out_specs=pl.BlockSpec((1, T, Oh), lambda b: (b, 0, 0)),
        out_shape=jax.ShapeDtypeStruct((B, T, Oh), jnp.float32),
    )(xf, wft, wht)
    return out.reshape(B, H, Wp, Oh)[:, :, :W, :]


def _mm_kernel(x_ref, w_ref, o_ref):
    o_ref[0] = _dot(x_ref[0], w_ref[...])


def _mm_add_kernel(x_ref, w_ref, u_ref, o_ref):
    o_ref[0] = _dot(x_ref[0], w_ref[...]) + u_ref[0]


def _conv1x1(x, w, u=None, nt=1):
    B, H, W, Cin = x.shape
    wt = jnp.transpose(w[:, :, 0, 0], (1, 0))
    O = wt.shape[1]
    N = H * W
    Nt = N // nt
    xf = x.reshape(B, N, Cin)
    in_specs = [
        pl.BlockSpec((1, Nt, Cin), lambda b, i: (b, i, 0)),
        pl.BlockSpec((Cin, O), lambda b, i: (0, 0)),
    ]
    args = [xf, wt]
    if u is None:
        kfn = _mm_kernel
    else:
        kfn = _mm_add_kernel
        in_specs.append(pl.BlockSpec((1, Nt, O), lambda b, i: (b, i, 0)))
        args.append(u.reshape(B, N, O))
    out = pl.pallas_call(
        kfn,
        grid=(B, nt),
        in_specs=in_specs,
        out_specs=pl.BlockSpec((1, Nt, O), lambda b, i: (b, i, 0)),
        out_shape=jax.ShapeDtypeStruct((B, N, O), jnp.float32),
    )(*args)
    return out.reshape(B, H, W, O)


NUM_CLASSES = 80
NUM_ANCHORS = 3


def _up2(u):
    u = jnp.repeat(u, 2, axis=1)
    return jnp.repeat(u, 2, axis=2)


def _head_out(o_nhwc):
    B, G, _, C = o_nhwc.shape
    o = o_nhwc.reshape(B, G, G, NUM_ANCHORS, 5 + NUM_CLASSES)
    return jnp.transpose(o, (0, 3, 1, 2, 4))


def kernel(x, W1, W2, W3, W4, W5, L3, L4, L5, F3, F4, F5, H3, H4, H5):
    xh = _to_nhwc(x)
    c1 = _conv3x3_s2(xh, W1, R=8)
    c2 = _conv3x3_s2(c1, W2, R=8)
    c3 = _conv3x3_s2(c2, W3, R=13)
    c4 = _conv3x3_s2(c3, W4, R=13)
    c5 = _conv3x3_s2(c4, W5, R=13)
    p5 = _conv1x1(c5, L5)
    p4 = _conv1x1(c4, L4, u=_up2(p5))
    p3 = _conv1x1(c3, L3, u=_up2(p4), nt=2)
    o3 = _conv3x3_s1_head(p3, F3, H3)
    o4 = _conv3x3_s1_head(p4, F4, H4)
    o5 = _conv3x3_s1_head(p5, F5, H5)
    return (_head_out(o3), _head_out(o4), _head_out(o5))
